# baseline (speedup 1.0000x reference)
import jax
import jax.numpy as jnp
from jax.experimental import pallas as pl
from jax.experimental.pallas import tpu as pltpu


_NCHUNK = 32


def _copy_body(src, dst, sem):
    rows = src.shape[0]
    step = rows // _NCHUNK
    copies = [
        pltpu.make_async_copy(
            src.at[pl.ds(i * step, step)], dst.at[pl.ds(i * step, step)], sem
        )
        for i in range(_NCHUNK)
    ]
    for c in copies:
        c.start()
    for c in copies:
        c.wait()


def kernel(x, emb_weight):
    sl = x.shape[1]
    dim = emb_weight.shape[1]
    return pl.pallas_call(
        _copy_body,
        out_shape=jax.ShapeDtypeStruct((sl, dim), emb_weight.dtype),
        in_specs=[pl.BlockSpec(memory_space=pl.ANY)],
        out_specs=pl.BlockSpec(memory_space=pl.ANY),
        scratch_shapes=[pltpu.SemaphoreType.DMA],
    )(emb_weight[:sl])


# speedup vs baseline: 47.1878x; 47.1878x over previous
import jax
import jax.numpy as jnp
from jax.experimental import pallas as pl
from jax.experimental.pallas import tpu as pltpu


_BLOCK_ROWS = 512


def _copy_body(src, dst):
    dst[...] = src[...]


def kernel(x, emb_weight):
    sl = x.shape[1]
    dim = emb_weight.shape[1]
    grid = sl // _BLOCK_ROWS
    return pl.pallas_call(
        _copy_body,
        out_shape=jax.ShapeDtypeStruct((sl, dim), emb_weight.dtype),
        grid=(grid,),
        in_specs=[pl.BlockSpec((_BLOCK_ROWS, dim), lambda i: (i, 0))],
        out_specs=pl.BlockSpec((_BLOCK_ROWS, dim), lambda i: (i, 0)),
    )(emb_weight[:sl])


# speedup vs baseline: 48.6045x; 1.0300x over previous
import jax
import jax.numpy as jnp
from jax.experimental import pallas as pl
from jax.experimental.pallas import tpu as pltpu


_BLOCK_ROWS = 1024


def _copy_body(src, dst):
    dst[...] = src[...]


def kernel(x, emb_weight):
    sl = x.shape[1]
    dim = emb_weight.shape[1]
    grid = sl // _BLOCK_ROWS
    return pl.pallas_call(
        _copy_body,
        out_shape=jax.ShapeDtypeStruct((sl, dim), emb_weight.dtype),
        grid=(grid,),
        in_specs=[pl.BlockSpec((_BLOCK_ROWS, dim), lambda i: (i, 0))],
        out_specs=pl.BlockSpec((_BLOCK_ROWS, dim), lambda i: (i, 0)),
    )(emb_weight[:sl])
